# baseline (speedup 1.0000x reference)
import functools

import jax
import jax.numpy as jnp
from jax import lax
from jax.experimental import pallas as pl
from jax.experimental.pallas import tpu as pltpu
from jax.experimental.pallas import tpu_sc as plsc

N = 10000
IN_DIM = 128
HID_DIM = 128
EMB_DIM = 64
N_CLASSES = 16
E = 320000

NC = 2
NS = 16
NW = NC * NS
CH = 80
NCHD = E // NW // CH
NCHA = E // NS // CH
IR = 1000
NIT = N // IR
NB = 5

_HIGHEST = jax.lax.Precision.HIGHEST

_sc_mesh = plsc.VectorSubcoreMesh(
    core_axis_name="c", subcore_axis_name="s", num_cores=NC, num_subcores=NS
)
_sc_params = pltpu.CompilerParams(use_tc_tiling_on_sc=False)


@functools.partial(
    pl.kernel,
    out_type=jax.ShapeDtypeStruct((NC, N, 1), jnp.float32),
    mesh=_sc_mesh,
    scratch_types=[
        pltpu.VMEM((NCHD, CH), jnp.int32),
        pltpu.VMEM((CH, 1), jnp.float32),
        pltpu.VMEM_SHARED((N, 1), jnp.float32),
        pltpu.SemaphoreType.DMA,
    ],
    compiler_params=_sc_params,
)
def _deg_kernel(dst_hbm, z_hbm, o_hbm, out_hbm, dst_v, ones_v, acc, sem):
    cid = lax.axis_index("c")
    sid = lax.axis_index("s")
    wid = cid * NS + sid
    rs = pl.ds(sid * IR, IR)

    @pl.when(jnp.logical_and(cid == 0, sid < NIT))
    def _():
        pltpu.sync_copy(z_hbm.at[rs], acc.at[rs])

    @pl.when(jnp.logical_and(cid == 1, sid < NIT))
    def _():
        pltpu.sync_copy(o_hbm.at[rs], acc.at[rs])

    pltpu.sync_copy(dst_hbm.at[wid], dst_v)
    pltpu.sync_copy(o_hbm.at[pl.ds(0, CH)], ones_v)
    plsc.subcore_barrier()

    def body(j, c):
        pltpu.async_copy(ones_v, acc.at[dst_v.at[j]], sem, add=True)
        return c

    lax.fori_loop(0, NCHD, body, 0)

    def drain(j, c):
        pltpu.make_async_copy(ones_v, acc.at[dst_v.at[0]], sem).wait()
        return c

    lax.fori_loop(0, NCHD, drain, 0)
    plsc.subcore_barrier()

    @pl.when(sid < NIT)
    def _():
        pltpu.sync_copy(acc.at[rs], out_hbm.at[cid, rs])


def _make_agg(Dh):
    @functools.partial(
        pl.kernel,
        out_type=jax.ShapeDtypeStruct((N, NC, Dh), jnp.float32),
        mesh=_sc_mesh,
        scratch_types=[
            pltpu.VMEM((NCHA, CH), jnp.int32),
            pltpu.VMEM((NCHA, CH), jnp.int32),
            pltpu.VMEM((NB, CH, Dh), jnp.float32),
            pltpu.VMEM_SHARED((N, Dh), jnp.float32),
        ] + [pltpu.SemaphoreType.DMA] * (2 * NB),
        compiler_params=_sc_params,
    )
    def agg(y2d, y3d, idx2_hbm, dst_hbm, out_hbm,
            src_v, dst_v, rows_v, acc, *sems):
        gsem = sems[:NB]
        ssem = sems[NB:]
        cid = lax.axis_index("c")
        sid = lax.axis_index("s")
        rs = pl.ds(sid * IR, IR)

        def start_g(j, b):
            pltpu.async_copy(y2d.at[src_v.at[j]], rows_v.at[b], gsem[b])

        def wait_g(b):
            pltpu.make_async_copy(
                y2d.at[src_v.at[0]], rows_v.at[b], gsem[b]).wait()

        def start_s(j, b):
            pltpu.async_copy(rows_v.at[b], acc.at[dst_v.at[j]], ssem[b],
                             add=True)

        def wait_s(b):
            pltpu.make_async_copy(
                rows_v.at[b], acc.at[dst_v.at[0]], ssem[b]).wait()

        @pl.when(sid < NIT)
        def _():
            pltpu.sync_copy(y3d.at[rs, cid], acc.at[rs])

        pltpu.sync_copy(idx2_hbm.at[cid, sid], src_v)
        pltpu.sync_copy(dst_hbm.at[sid], dst_v)
        plsc.subcore_barrier()

        for b in range(NB - 1):
            start_g(b, b)

        def outer(t, c):
            for k in range(NB):
                j = t * NB + k
                wait_g(k)
                start_s(j, k)
                nb = (k + NB - 1) % NB

                @pl.when(j > 0)
                def _():
                    wait_s(nb)

                @pl.when(j + NB - 1 < NCHA)
                def _():
                    start_g(j + NB - 1, nb)
            return c

        lax.fori_loop(0, NCHA // NB, outer, 0)
        wait_s((NCHA - 1) % NB)
        plsc.subcore_barrier()

        @pl.when(sid < NIT)
        def _():
            pltpu.sync_copy(acc.at[rs], out_hbm.at[rs, cid])

    return agg


_agg1 = _make_agg(HID_DIM // 2)
_agg2 = _make_agg(EMB_DIM // 2)


R = 2000
GRID = N // R


def _tc1_body(x_ref, w_ref, dp_ref, y_ref, dinv_ref):
    xw = lax.dot_general(
        x_ref[...], w_ref[...], (((1,), (0,)), ((), ())),
        preferred_element_type=jnp.float32, precision=_HIGHEST,
    )
    dinv = lax.rsqrt(dp_ref[0] + dp_ref[1])
    dinv_ref[...] = dinv
    y_ref[...] = xw * dinv


_tc1 = pl.pallas_call(
    _tc1_body,
    grid=(GRID,),
    in_specs=[
        pl.BlockSpec((R, IN_DIM), lambda i: (i, 0)),
        pl.BlockSpec((IN_DIM, HID_DIM), lambda i: (0, 0)),
        pl.BlockSpec((NC, R, 1), lambda i: (0, i, 0)),
    ],
    out_specs=[
        pl.BlockSpec((R, HID_DIM), lambda i: (i, 0)),
        pl.BlockSpec((R, 1), lambda i: (i, 0)),
    ],
    out_shape=[
        jax.ShapeDtypeStruct((N, HID_DIM), jnp.float32),
        jax.ShapeDtypeStruct((N, 1), jnp.float32),
    ],
)


def _tc2_body(p_ref, dinv_ref, b1_ref, g1_ref, bt_ref, w2_ref, y2_ref):
    dinv = dinv_ref[...]
    pre = p_ref[...] * dinv + b1_ref[...]
    mu = jnp.mean(pre, axis=-1, keepdims=True)
    cen = pre - mu
    var = jnp.mean(cen * cen, axis=-1, keepdims=True)
    h = cen * lax.rsqrt(var + 1e-5) * g1_ref[...] + bt_ref[...]
    h = jnp.maximum(h, 0.0)
    hw = lax.dot_general(
        h, w2_ref[...], (((1,), (0,)), ((), ())),
        preferred_element_type=jnp.float32, precision=_HIGHEST,
    )
    y2_ref[...] = hw * dinv


_tc2 = pl.pallas_call(
    _tc2_body,
    grid=(GRID,),
    in_specs=[
        pl.BlockSpec((R, HID_DIM), lambda i: (i, 0)),
        pl.BlockSpec((R, 1), lambda i: (i, 0)),
        pl.BlockSpec((1, HID_DIM), lambda i: (0, 0)),
        pl.BlockSpec((1, HID_DIM), lambda i: (0, 0)),
        pl.BlockSpec((1, HID_DIM), lambda i: (0, 0)),
        pl.BlockSpec((HID_DIM, EMB_DIM), lambda i: (0, 0)),
    ],
    out_specs=pl.BlockSpec((R, EMB_DIM), lambda i: (i, 0)),
    out_shape=jax.ShapeDtypeStruct((N, EMB_DIM), jnp.float32),
)


def _tc3_body(p_ref, dinv_ref, b2_ref, wc_ref, bc_ref, out_ref):
    emb = p_ref[...] * dinv_ref[...] + b2_ref[...]
    out_ref[...] = lax.dot_general(
        emb, wc_ref[...], (((1,), (0,)), ((), ())),
        preferred_element_type=jnp.float32, precision=_HIGHEST,
    ) + bc_ref[...]


_tc3 = pl.pallas_call(
    _tc3_body,
    grid=(GRID,),
    in_specs=[
        pl.BlockSpec((R, EMB_DIM), lambda i: (i, 0)),
        pl.BlockSpec((R, 1), lambda i: (i, 0)),
        pl.BlockSpec((1, EMB_DIM), lambda i: (0, 0)),
        pl.BlockSpec((EMB_DIM, N_CLASSES), lambda i: (0, 0)),
        pl.BlockSpec((1, N_CLASSES), lambda i: (0, 0)),
    ],
    out_specs=pl.BlockSpec((R, N_CLASSES), lambda i: (i, 0)),
    out_shape=jax.ShapeDtypeStruct((N, N_CLASSES), jnp.float32),
)


def kernel(x, edge_index, W1, b1, g1, beta1, W2, b2, Wc, bc):
    src = edge_index[0].astype(jnp.int32)
    dst = edge_index[1].astype(jnp.int32)
    src_rs = src.reshape(NS, NCHA, CH)
    idx2 = jnp.stack([src_rs * 2, src_rs * 2 + 1])
    dst_rs = dst.reshape(NS, NCHA, CH)
    dst_deg = dst.reshape(NW, NCHD, CH)

    zeros1 = jnp.zeros((N, 1), jnp.float32)
    ones1 = jnp.ones((N, 1), jnp.float32)
    degp = _deg_kernel(dst_deg, zeros1, ones1)
    y1, dinv = _tc1(x, W1, degp)

    p1 = _agg1(y1.reshape(2 * N, HID_DIM // 2),
               y1.reshape(N, 2, HID_DIM // 2),
               idx2, dst_rs).reshape(N, HID_DIM)
    y2 = _tc2(p1, dinv, b1.reshape(1, -1), g1.reshape(1, -1),
              beta1.reshape(1, -1), W2)

    p2 = _agg2(y2.reshape(2 * N, EMB_DIM // 2),
               y2.reshape(N, 2, EMB_DIM // 2),
               idx2, dst_rs).reshape(N, EMB_DIM)
    logits = _tc3(p2, dinv, b2.reshape(1, -1), Wc, bc.reshape(1, -1))
    return logits


# speedup vs baseline: 1.3658x; 1.3658x over previous
import functools

import jax
import jax.numpy as jnp
from jax import lax
from jax.experimental import pallas as pl
from jax.experimental.pallas import tpu as pltpu
from jax.experimental.pallas import tpu_sc as plsc

N = 10000
IN_DIM = 128
HID_DIM = 128
EMB_DIM = 64
N_CLASSES = 16
E = 320000

NC = 2
NS = 16
NW = NC * NS
CH = 80
NCHD = E // NW // CH
NCHA = E // NS // CH
IR = 1000
NIT = N // IR
NB = 5
HH = HID_DIM // 2
EH = EMB_DIM // 2

_HIGHEST = jax.lax.Precision.HIGHEST

_sc_mesh = plsc.VectorSubcoreMesh(
    core_axis_name="c", subcore_axis_name="s", num_cores=NC, num_subcores=NS
)
_sc_params = pltpu.CompilerParams(use_tc_tiling_on_sc=False)


@functools.partial(
    pl.kernel,
    out_type=jax.ShapeDtypeStruct((NC, N, 1), jnp.float32),
    mesh=_sc_mesh,
    scratch_types=[
        pltpu.VMEM((NCHD, CH), jnp.int32),
        pltpu.VMEM((CH, 1), jnp.float32),
        pltpu.VMEM_SHARED((N, 1), jnp.float32),
        pltpu.SemaphoreType.DMA,
    ],
    compiler_params=_sc_params,
)
def _deg_kernel(dst_hbm, z_hbm, o_hbm, out_hbm, dst_v, ones_v, acc, sem):
    cid = lax.axis_index("c")
    sid = lax.axis_index("s")
    wid = cid * NS + sid
    rs = pl.ds(sid * IR, IR)

    @pl.when(jnp.logical_and(cid == 0, sid < NIT))
    def _():
        pltpu.sync_copy(z_hbm.at[rs], acc.at[rs])

    @pl.when(jnp.logical_and(cid == 1, sid < NIT))
    def _():
        pltpu.sync_copy(o_hbm.at[rs], acc.at[rs])

    pltpu.sync_copy(dst_hbm.at[wid], dst_v)
    pltpu.sync_copy(o_hbm.at[pl.ds(0, CH)], ones_v)
    plsc.subcore_barrier()

    def body(j, c):
        pltpu.async_copy(ones_v, acc.at[dst_v.at[j]], sem, add=True)
        return c

    lax.fori_loop(0, NCHD, body, 0)

    def drain(j, c):
        pltpu.make_async_copy(ones_v, acc.at[dst_v.at[0]], sem).wait()
        return c

    lax.fori_loop(0, NCHD, drain, 0)
    plsc.subcore_barrier()

    @pl.when(sid < NIT)
    def _():
        pltpu.sync_copy(acc.at[rs], out_hbm.at[cid, rs])


def _make_agg(Dh):
    @functools.partial(
        pl.kernel,
        out_type=jax.ShapeDtypeStruct((NC, N, Dh), jnp.float32),
        mesh=_sc_mesh,
        scratch_types=[
            pltpu.VMEM((NCHA, CH), jnp.int32),
            pltpu.VMEM((NCHA, CH), jnp.int32),
            pltpu.VMEM((NB, CH, Dh), jnp.float32),
            pltpu.VMEM_SHARED((N, Dh), jnp.float32),
        ] + [pltpu.SemaphoreType.DMA] * (2 * NB),
        compiler_params=_sc_params,
    )
    def agg(y_hbm, src_hbm, dst_hbm, out_hbm,
            src_v, dst_v, rows_v, acc, *sems):
        gsem = sems[:NB]
        ssem = sems[NB:]
        cid = lax.axis_index("c")
        sid = lax.axis_index("s")
        rs = pl.ds(sid * IR, IR)
        ytab = y_hbm.at[cid]

        def start_g(j, b):
            pltpu.async_copy(ytab.at[src_v.at[j]], rows_v.at[b], gsem[b])

        def wait_g(b):
            pltpu.make_async_copy(
                ytab.at[src_v.at[0]], rows_v.at[b], gsem[b]).wait()

        def start_s(j, b):
            pltpu.async_copy(rows_v.at[b], acc.at[dst_v.at[j]], ssem[b],
                             add=True)

        def wait_s(b):
            pltpu.make_async_copy(
                rows_v.at[b], acc.at[dst_v.at[0]], ssem[b]).wait()

        @pl.when(sid < NIT)
        def _():
            pltpu.sync_copy(ytab.at[rs], acc.at[rs])

        pltpu.sync_copy(src_hbm.at[sid], src_v)
        pltpu.sync_copy(dst_hbm.at[sid], dst_v)
        plsc.subcore_barrier()

        for b in range(NB - 1):
            start_g(b, b)

        def outer(t, c):
            for k in range(NB):
                j = t * NB + k
                wait_g(k)
                start_s(j, k)
                nb = (k + NB - 1) % NB

                @pl.when(j > 0)
                def _():
                    wait_s(nb)

                @pl.when(j + NB - 1 < NCHA)
                def _():
                    start_g(j + NB - 1, nb)
            return c

        lax.fori_loop(0, NCHA // NB, outer, 0)
        wait_s((NCHA - 1) % NB)
        plsc.subcore_barrier()

        @pl.when(sid < NIT)
        def _():
            pltpu.sync_copy(acc.at[rs], out_hbm.at[cid, rs])

    return agg


_agg1 = _make_agg(HH)
_agg2 = _make_agg(EH)


R = 2000
GRID = N // R


def _tc1_body(x_ref, w_ref, dp_ref, y_ref, dinv_ref):
    x = x_ref[...]
    dinv = lax.rsqrt(dp_ref[0] + dp_ref[1])
    dinv_ref[...] = dinv
    for h in range(NC):
        xw = lax.dot_general(
            x, w_ref[:, h * HH:(h + 1) * HH], (((1,), (0,)), ((), ())),
            preferred_element_type=jnp.float32, precision=_HIGHEST,
        )
        y_ref[h] = xw * dinv


_tc1 = pl.pallas_call(
    _tc1_body,
    grid=(GRID,),
    in_specs=[
        pl.BlockSpec((R, IN_DIM), lambda i: (i, 0)),
        pl.BlockSpec((IN_DIM, HID_DIM), lambda i: (0, 0)),
        pl.BlockSpec((NC, R, 1), lambda i: (0, i, 0)),
    ],
    out_specs=[
        pl.BlockSpec((NC, R, HH), lambda i: (0, i, 0)),
        pl.BlockSpec((R, 1), lambda i: (i, 0)),
    ],
    out_shape=[
        jax.ShapeDtypeStruct((NC, N, HH), jnp.float32),
        jax.ShapeDtypeStruct((N, 1), jnp.float32),
    ],
)


def _tc2_body(p_ref, dinv_ref, b1_ref, g1_ref, bt_ref, w2_ref, y2_ref):
    dinv = dinv_ref[...]
    pre = [p_ref[h] * dinv + b1_ref[:, h * HH:(h + 1) * HH] for h in range(2)]
    mu = (jnp.sum(pre[0], -1, keepdims=True)
          + jnp.sum(pre[1], -1, keepdims=True)) * (1.0 / HID_DIM)
    cen = [pre[h] - mu for h in range(2)]
    var = (jnp.sum(cen[0] * cen[0], -1, keepdims=True)
           + jnp.sum(cen[1] * cen[1], -1, keepdims=True)) * (1.0 / HID_DIM)
    rstd = lax.rsqrt(var + 1e-5)
    hs = [
        jnp.maximum(cen[h] * rstd * g1_ref[:, h * HH:(h + 1) * HH]
                    + bt_ref[:, h * HH:(h + 1) * HH], 0.0)
        for h in range(2)
    ]
    for e in range(NC):
        hw = None
        for h in range(2):
            d = lax.dot_general(
                hs[h], w2_ref[h * HH:(h + 1) * HH, e * EH:(e + 1) * EH],
                (((1,), (0,)), ((), ())),
                preferred_element_type=jnp.float32, precision=_HIGHEST,
            )
            hw = d if hw is None else hw + d
        y2_ref[e] = hw * dinv


_tc2 = pl.pallas_call(
    _tc2_body,
    grid=(GRID,),
    in_specs=[
        pl.BlockSpec((NC, R, HH), lambda i: (0, i, 0)),
        pl.BlockSpec((R, 1), lambda i: (i, 0)),
        pl.BlockSpec((1, HID_DIM), lambda i: (0, 0)),
        pl.BlockSpec((1, HID_DIM), lambda i: (0, 0)),
        pl.BlockSpec((1, HID_DIM), lambda i: (0, 0)),
        pl.BlockSpec((HID_DIM, EMB_DIM), lambda i: (0, 0)),
    ],
    out_specs=pl.BlockSpec((NC, R, EH), lambda i: (0, i, 0)),
    out_shape=jax.ShapeDtypeStruct((NC, N, EH), jnp.float32),
)


def _tc3_body(p_ref, dinv_ref, b2_ref, wc_ref, bc_ref, out_ref):
    dinv = dinv_ref[...]
    acc = None
    for h in range(2):
        emb = p_ref[h] * dinv + b2_ref[:, h * EH:(h + 1) * EH]
        d = lax.dot_general(
            emb, wc_ref[h * EH:(h + 1) * EH, :], (((1,), (0,)), ((), ())),
            preferred_element_type=jnp.float32, precision=_HIGHEST,
        )
        acc = d if acc is None else acc + d
    out_ref[...] = acc + bc_ref[...]


_tc3 = pl.pallas_call(
    _tc3_body,
    grid=(GRID,),
    in_specs=[
        pl.BlockSpec((NC, R, EH), lambda i: (0, i, 0)),
        pl.BlockSpec((R, 1), lambda i: (i, 0)),
        pl.BlockSpec((1, EMB_DIM), lambda i: (0, 0)),
        pl.BlockSpec((EMB_DIM, N_CLASSES), lambda i: (0, 0)),
        pl.BlockSpec((1, N_CLASSES), lambda i: (0, 0)),
    ],
    out_specs=pl.BlockSpec((R, N_CLASSES), lambda i: (i, 0)),
    out_shape=jax.ShapeDtypeStruct((N, N_CLASSES), jnp.float32),
)


def kernel(x, edge_index, W1, b1, g1, beta1, W2, b2, Wc, bc):
    src = edge_index[0].astype(jnp.int32)
    dst = edge_index[1].astype(jnp.int32)
    src_agg = src.reshape(NS, NCHA, CH)
    dst_agg = dst.reshape(NS, NCHA, CH)
    dst_deg = dst.reshape(NW, NCHD, CH)

    zeros1 = jnp.zeros((N, 1), jnp.float32)
    ones1 = jnp.ones((N, 1), jnp.float32)
    degp = _deg_kernel(dst_deg, zeros1, ones1)
    y1, dinv = _tc1(x, W1, degp)

    p1 = _agg1(y1, src_agg, dst_agg)
    y2 = _tc2(p1, dinv, b1.reshape(1, -1), g1.reshape(1, -1),
              beta1.reshape(1, -1), W2)

    p2 = _agg2(y2, src_agg, dst_agg)
    logits = _tc3(p2, dinv, b2.reshape(1, -1), Wc, bc.reshape(1, -1))
    return logits
